# baseline (speedup 1.0000x reference)
import functools

import jax
import jax.numpy as jnp
from jax import lax
from jax.experimental import pallas as pl
from jax.experimental.pallas import tpu as pltpu
from jax.experimental.pallas import tpu_sc as plsc

N = 50000
E = 800000
VHS = 100
DP = 128
CB = 32
NCB = DP // CB
NSV = 5000
NROUNDS = 4

BN = 1000
CHUNK = 128
NS = 16
NC = 2

f32 = jnp.float32
i32 = jnp.int32

EROWS = 6272
EPAD = EROWS * CHUNK - E
TROWS = EROWS // NS
SB = 8
NSB = TROWS // SB
NP = 50176
ROWS_PER_TILE = NP // NS
WCHUNK = 112


def _full(shape):
    return pl.BlockSpec(shape, lambda i: (0,) * len(shape))


def _embed_body(x_ref, tv_ref, o_ref):
    o_ref[...] = jnp.dot(x_ref[...], tv_ref[...], preferred_element_type=f32)


def _embed(x_pad, tv):
    return pl.pallas_call(
        _embed_body,
        grid=(N // BN,),
        in_specs=[pl.BlockSpec((BN, DP), lambda i: (i, 0)), _full((DP, DP))],
        out_specs=pl.BlockSpec((BN, DP), lambda i: (i, 0)),
        out_shape=jax.ShapeDtypeStruct((N, DP), f32),
    )(x_pad, tv)


def _mlp_body(x_ref, w1, b1, w2, b2, w3, b3, o_ref):
    x = x_ref[...]
    h = jnp.maximum(jnp.dot(x, w1[...], preferred_element_type=f32) + b1[...], 0.0)
    h = jnp.maximum(jnp.dot(h, w2[...], preferred_element_type=f32) + b2[...], 0.0)
    o_ref[...] = jnp.dot(h, w3[...], preferred_element_type=f32) + b3[...]


def _mlp(xs, p):
    return pl.pallas_call(
        _mlp_body,
        grid=(N // BN,),
        in_specs=[pl.BlockSpec((BN, DP), lambda i: (i, 0)),
                  _full((DP, DP)), _full((1, DP)),
                  _full((DP, DP)), _full((1, DP)),
                  _full((DP, DP)), _full((1, DP))],
        out_specs=pl.BlockSpec((BN, DP), lambda i: (i, 0)),
        out_shape=jax.ShapeDtypeStruct((N, DP), f32),
    )(xs, p["w1"], p["b1"], p["w2"], p["b2"], p["w3"], p["b3"])


def _gru_body(msg4_ref, den_ref, h_ref, wi_ref, wh_ref, bi_ref, bh_ref, o_ref):
    msg = jnp.concatenate(
        [msg4_ref[0], msg4_ref[1], msg4_ref[2], msg4_ref[3]], axis=-1)
    xin = msg / den_ref[...]
    h = h_ref[...]
    gi = jnp.dot(xin, wi_ref[...], preferred_element_type=f32) + bi_ref[...]
    gh = jnp.dot(h, wh_ref[...], preferred_element_type=f32) + bh_ref[...]
    r = jax.nn.sigmoid(gi[:, 0:DP] + gh[:, 0:DP])
    z = jax.nn.sigmoid(gi[:, DP:2 * DP] + gh[:, DP:2 * DP])
    n = jnp.tanh(gi[:, 2 * DP:3 * DP] + r * gh[:, 2 * DP:3 * DP])
    o_ref[...] = (1.0 - z) * n + z * h


def _gru(msg4, den, h, p):
    return pl.pallas_call(
        _gru_body,
        grid=(N // BN,),
        in_specs=[pl.BlockSpec((NCB, BN, CB), lambda i: (0, i, 0)),
                  pl.BlockSpec((BN, 1), lambda i: (i, 0)),
                  pl.BlockSpec((BN, DP), lambda i: (i, 0)),
                  _full((DP, 3 * DP)), _full((DP, 3 * DP)),
                  _full((1, 3 * DP)), _full((1, 3 * DP))],
        out_specs=pl.BlockSpec((BN, DP), lambda i: (i, 0)),
        out_shape=jax.ShapeDtypeStruct((N, DP), f32),
    )(msg4, den, h, p["wi"], p["wh"], p["bi"], p["bh"])


def _head_body(x_ref, w1, b1, w2, b2, o_ref):
    h = jnp.maximum(jnp.dot(x_ref[...], w1[...], preferred_element_type=f32)
                    + b1[...], 0.0)
    o_ref[...] = jnp.dot(h, w2[...], preferred_element_type=f32) + b2[...]


def _head(xs, p):
    return pl.pallas_call(
        _head_body,
        grid=(NSV // BN,),
        in_specs=[pl.BlockSpec((BN, DP), lambda i: (i, 0)),
                  _full((DP, DP)), _full((1, DP)),
                  _full((DP, DP)), _full((1, DP))],
        out_specs=pl.BlockSpec((BN, DP), lambda i: (i, 0)),
        out_shape=jax.ShapeDtypeStruct((NSV, DP), f32),
    )(xs, p["w1"], p["b1"], p["w2"], p["b2"])


@functools.cache
def _mesh():
    return plsc.VectorSubcoreMesh(core_axis_name="c", subcore_axis_name="s")


@functools.cache
def _build_segsum():
    return functools.partial(
        pl.kernel,
        out_type=jax.ShapeDtypeStruct((NCB, NP, CB), f32),
        mesh=_mesh(),
        scratch_types=[
            pltpu.VMEM((2, SB, CHUNK), i32),
            pltpu.VMEM((2, SB, CHUNK), i32),
            pltpu.VMEM((2, SB, CHUNK), i32),
            pltpu.VMEM((2, CHUNK, CB), f32),
            pltpu.VMEM((WCHUNK, CB), f32),
            pltpu.VMEM((WCHUNK, CB), f32),
            pltpu.VMEM_SHARED((NP, CB), f32),
            pltpu.SemaphoreType.DMA,
            pltpu.SemaphoreType.DMA,
            pltpu.SemaphoreType.DMA,
        ],
        compiler_params=pltpu.CompilerParams(use_tc_tiling_on_sc=False),
    )(_segsum_body)


def _segsum(m2, gsrc2, sdst2):
    return _build_segsum()(m2, gsrc2, sdst2)


def _segsum_body(m2, gsrc2, sdst2, out, sidx, didx, ridx, rows, zbuf, obuf,
                 acc, gsem, isem, ssem):
    c = lax.axis_index("c")
    s = lax.axis_index("s")
    rowbase = s * TROWS
    rbase = s * ROWS_PER_TILE

    def _z(i, _):
        zbuf[i // 2, pl.ds((i % 2) * 16, 16)] = jnp.zeros((16,), f32)
        return _
    lax.fori_loop(0, WCHUNK * 2, _z, None)

    def idx_fire(i, u):
        r0 = jnp.minimum(rowbase + i * SB, EROWS - SB)
        pltpu.async_copy(gsrc2.at[pl.ds(r0, SB)], sidx.at[u], isem)
        pltpu.async_copy(sdst2.at[pl.ds(r0, SB)], didx.at[u], isem)

    def idx_wait(u):
        pltpu.make_async_copy(gsrc2.at[pl.ds(0, SB)], sidx.at[u], isem).wait()
        pltpu.make_async_copy(sdst2.at[pl.ds(0, SB)], didx.at[u], isem).wait()

    def ridx_compute(u, cbi):
        for k in range(SB):
            for q in range(CHUNK // 16):
                v = sidx[u, k, pl.ds(q * 16, 16)]
                ridx[u, k, pl.ds(q * 16, 16)] = v * NCB + cbi

    def gfire(u, k, p):
        pltpu.async_copy(m2.at[ridx.at[u, k]], rows.at[p], gsem)

    def gwait(p):
        pltpu.make_async_copy(m2.at[pl.ds(0, CHUNK)], rows.at[p], gsem).wait()

    def swait(p):
        pltpu.make_async_copy(m2.at[pl.ds(0, CHUNK)], rows.at[p], ssem).wait()

    def process_sblk(i, u, cbi, first=False):
        un = 1 - u

        @pl.when(i < NSB - 1)
        def _():
            idx_wait(un)
            ridx_compute(un, cbi)
        for k in range(SB):
            p = k % 2
            gwait(p)
            pltpu.async_copy(rows.at[p], acc.at[didx.at[u, k]], ssem,
                             add=True)
            if not (first and k == 0):
                swait(1 - p)
            if k < SB - 1:
                gfire(u, k + 1, 1 - p)
            else:
                @pl.when(i < NSB - 1)
                def _():
                    gfire(un, 0, 1 - p)

        @pl.when(i + 2 < NSB)
        def _():
            idx_fire(i + 2, u)

    def _pass(pp):
        cbi = c * 2 + pp

        def _zc(k, _):
            pltpu.sync_copy(zbuf, acc.at[pl.ds(rbase + k * WCHUNK, WCHUNK)])
            return _
        lax.fori_loop(0, ROWS_PER_TILE // WCHUNK, _zc, None)
        plsc.subcore_barrier()

        idx_fire(0, 0)
        idx_fire(1, 1)
        idx_wait(0)
        ridx_compute(0, cbi)
        gfire(0, 0, 0)
        process_sblk(jnp.int32(0), 0, cbi, first=True)

        def _sbpair(i2, _):
            process_sblk(2 * i2 + 1, 1, cbi)
            process_sblk(2 * i2 + 2, 0, cbi)
            return _
        lax.fori_loop(0, (NSB - 1) // 2, _sbpair, None)
        swait(1)
        plsc.subcore_barrier()

        def _wc(k, _):
            r0 = rbase + k * WCHUNK
            pltpu.sync_copy(acc.at[pl.ds(r0, WCHUNK)], obuf)
            pltpu.sync_copy(obuf, out.at[cbi, pl.ds(r0, WCHUNK)])
            return _
        lax.fori_loop(0, ROWS_PER_TILE // WCHUNK, _wc, None)
        plsc.subcore_barrier()

    _pass(0)
    _pass(1)


_SV_PER_W = 200
_SV_WORKERS = NSV // _SV_PER_W


@functools.cache
def _build_sv_gather():
    return functools.partial(
        pl.kernel,
        out_type=jax.ShapeDtypeStruct((NSV, DP), f32),
        mesh=_mesh(),
        scratch_types=[
            pltpu.VMEM((_SV_PER_W,), i32),
            pltpu.VMEM((_SV_PER_W, DP), f32),
            pltpu.SemaphoreType.DMA,
        ],
    )(_sv_gather_body)


def _sv_gather(vs, sv):
    return _build_sv_gather()(vs, sv)


def _sv_gather_body(vs, sv, out, idxb, rowsb, sem):
    c = lax.axis_index("c")
    s = lax.axis_index("s")
    w = s * NC + c

    @pl.when(w < _SV_WORKERS)
    def _():
        base = w * _SV_PER_W
        pltpu.sync_copy(sv.at[pl.ds(base, _SV_PER_W)], idxb)
        pltpu.async_copy(vs.at[idxb], rowsb, sem).wait()
        pltpu.sync_copy(rowsb, out.at[pl.ds(base, _SV_PER_W)])


def _pad_wt(w, ki=VHS, ko=VHS):
    return jnp.zeros((DP, DP), f32).at[:ki, :ko].set(w.T[:ki, :ko])


def _pad_b(b):
    return jnp.zeros((1, DP), f32).at[0, :b.shape[0]].set(b)


def _prep_mlp(p):
    return {"w1": _pad_wt(p["w1"]), "b1": _pad_b(p["b1"]),
            "w2": _pad_wt(p["w2"]), "b2": _pad_b(p["b2"]),
            "w3": _pad_wt(p["w3"]), "b3": _pad_b(p["b3"])}


def _prep_gru(p):
    wi = jnp.zeros((DP, 3 * DP), f32)
    wh = jnp.zeros((DP, 3 * DP), f32)
    bi = jnp.zeros((1, 3 * DP), f32)
    bh = jnp.zeros((1, 3 * DP), f32)
    for g in range(3):
        wi = wi.at[:VHS, g * DP:g * DP + VHS].set(
            p["wi"][g * VHS:(g + 1) * VHS].T)
        wh = wh.at[:VHS, g * DP:g * DP + VHS].set(
            p["wh"][g * VHS:(g + 1) * VHS].T)
    for g in range(2):
        bi = bi.at[0, g * DP:g * DP + VHS].set(
            p["bi"][g * VHS:(g + 1) * VHS] + p["bh"][g * VHS:(g + 1) * VHS])
    bi = bi.at[0, 2 * DP:2 * DP + VHS].set(p["bi"][2 * VHS:3 * VHS])
    bh = bh.at[0, 2 * DP:2 * DP + VHS].set(p["bh"][2 * VHS:3 * VHS])
    return {"wi": wi, "wh": wh, "bi": bi, "bh": bh}


def kernel(x, ind, outd, params, sv_node, edge_index):
    nvt = x.shape[1]
    tv = params["type_w"] + params["type_b"]
    tv_pad = jnp.zeros((DP, DP), f32).at[:nvt, :VHS].set(tv)
    x_pad = jnp.zeros((N, DP), f32).at[:, :nvt].set(x)

    fmsg = _prep_mlp(params["fmsg"])
    bmsg = _prep_mlp(params["bmsg"])
    fgru = _prep_gru(params["fgru"])
    bgru = _prep_gru(params["bgru"])
    pre = {"w1": _pad_wt(params["pre"]["w1"], ki=VHS, ko=30),
           "b1": _pad_b(params["pre"]["b1"]),
           "w2": _pad_wt(params["pre"]["w2"], ki=30, ko=1),
           "b2": _pad_b(params["pre"]["b2"])}

    src = edge_index[0]
    dst = edge_index[1]
    gpad = jnp.arange(EPAD, dtype=i32) % 1024
    spad = N + jnp.arange(EPAD, dtype=i32) % 128
    gsrc_f = jnp.concatenate([src, gpad]).reshape(EROWS, CHUNK)
    sdst_f = jnp.concatenate([dst, spad]).reshape(EROWS, CHUNK)
    gsrc_b = jnp.concatenate([dst, gpad]).reshape(EROWS, CHUNK)
    sdst_b = jnp.concatenate([src, spad]).reshape(EROWS, CHUNK)
    ind_c = ind[:, None]
    outd_c = outd[:, None]

    vs = _embed(x_pad, tv_pad)
    for _ in range(NROUNDS):
        m = _mlp(vs, fmsg)
        raw4 = _segsum(m.reshape(N * NCB, CB), gsrc_f, sdst_f)
        vs = _gru(raw4, ind_c, vs, fgru)
        m = _mlp(vs, bmsg)
        raw4 = _segsum(m.reshape(N * NCB, CB), gsrc_b, sdst_b)
        vs = _gru(raw4, outd_c, vs, bgru)

    shortcut = _sv_gather(vs, sv_node)
    out = _head(shortcut, pre)
    return out[:, 0]


# speedup vs baseline: 1.2420x; 1.2420x over previous
import functools

import jax
import jax.numpy as jnp
from jax import lax
from jax.experimental import pallas as pl
from jax.experimental.pallas import tpu as pltpu
from jax.experimental.pallas import tpu_sc as plsc

N = 50000
E = 800000
VHS = 100
DP = 128
CB = 32
NCB = DP // CB
NSV = 5000
NROUNDS = 4

BN = 1000
CHUNK = 128
NS = 16
NC = 2

f32 = jnp.float32
i32 = jnp.int32

EROWS = 6272
EPAD = EROWS * CHUNK - E
TROWS = EROWS // NS
SB = 8
NSB = TROWS // SB
NP = 50176
ROWS_PER_TILE = NP // NS
WCHUNK = 112


def _full(shape):
    return pl.BlockSpec(shape, lambda i: (0,) * len(shape))


def _embed_body(x_ref, tv_ref, o_ref):
    o_ref[...] = jnp.dot(x_ref[...], tv_ref[...], preferred_element_type=f32)


def _embed(x_pad, tv):
    return pl.pallas_call(
        _embed_body,
        grid=(N // BN,),
        in_specs=[pl.BlockSpec((BN, DP), lambda i: (i, 0)), _full((DP, DP))],
        out_specs=pl.BlockSpec((BN, DP), lambda i: (i, 0)),
        out_shape=jax.ShapeDtypeStruct((N, DP), f32),
    )(x_pad, tv)


def _mlp_body(x_ref, w1, b1, w2, b2, w3, b3, o_ref):
    x = x_ref[...]
    h = jnp.maximum(jnp.dot(x, w1[...], preferred_element_type=f32) + b1[...], 0.0)
    h = jnp.maximum(jnp.dot(h, w2[...], preferred_element_type=f32) + b2[...], 0.0)
    o_ref[...] = jnp.dot(h, w3[...], preferred_element_type=f32) + b3[...]


def _mlp(xs, p):
    return pl.pallas_call(
        _mlp_body,
        grid=(N // BN,),
        in_specs=[pl.BlockSpec((BN, DP), lambda i: (i, 0)),
                  _full((DP, DP)), _full((1, DP)),
                  _full((DP, DP)), _full((1, DP)),
                  _full((DP, DP)), _full((1, DP))],
        out_specs=pl.BlockSpec((BN, DP), lambda i: (i, 0)),
        out_shape=jax.ShapeDtypeStruct((N, DP), f32),
    )(xs, p["w1"], p["b1"], p["w2"], p["b2"], p["w3"], p["b3"])


def _gru_body(msg4_ref, den_ref, h_ref, wi_ref, wh_ref, bi_ref, bh_ref, o_ref):
    msg = jnp.concatenate(
        [msg4_ref[0], msg4_ref[1], msg4_ref[2], msg4_ref[3]], axis=-1)
    xin = msg / den_ref[...]
    h = h_ref[...]
    gi = jnp.dot(xin, wi_ref[...], preferred_element_type=f32) + bi_ref[...]
    gh = jnp.dot(h, wh_ref[...], preferred_element_type=f32) + bh_ref[...]
    r = jax.nn.sigmoid(gi[:, 0:DP] + gh[:, 0:DP])
    z = jax.nn.sigmoid(gi[:, DP:2 * DP] + gh[:, DP:2 * DP])
    n = jnp.tanh(gi[:, 2 * DP:3 * DP] + r * gh[:, 2 * DP:3 * DP])
    o_ref[...] = (1.0 - z) * n + z * h


def _gru(msg4, den, h, p):
    return pl.pallas_call(
        _gru_body,
        grid=(N // BN,),
        in_specs=[pl.BlockSpec((NCB, BN, CB), lambda i: (0, i, 0)),
                  pl.BlockSpec((BN, 1), lambda i: (i, 0)),
                  pl.BlockSpec((BN, DP), lambda i: (i, 0)),
                  _full((DP, 3 * DP)), _full((DP, 3 * DP)),
                  _full((1, 3 * DP)), _full((1, 3 * DP))],
        out_specs=pl.BlockSpec((BN, DP), lambda i: (i, 0)),
        out_shape=jax.ShapeDtypeStruct((N, DP), f32),
    )(msg4, den, h, p["wi"], p["wh"], p["bi"], p["bh"])


def _head_body(x_ref, w1, b1, w2, b2, o_ref):
    h = jnp.maximum(jnp.dot(x_ref[...], w1[...], preferred_element_type=f32)
                    + b1[...], 0.0)
    o_ref[...] = jnp.dot(h, w2[...], preferred_element_type=f32) + b2[...]


def _head(xs, p):
    return pl.pallas_call(
        _head_body,
        grid=(NSV // BN,),
        in_specs=[pl.BlockSpec((BN, DP), lambda i: (i, 0)),
                  _full((DP, DP)), _full((1, DP)),
                  _full((DP, DP)), _full((1, DP))],
        out_specs=pl.BlockSpec((BN, DP), lambda i: (i, 0)),
        out_shape=jax.ShapeDtypeStruct((NSV, DP), f32),
    )(xs, p["w1"], p["b1"], p["w2"], p["b2"])


@functools.cache
def _mesh():
    return plsc.VectorSubcoreMesh(core_axis_name="c", subcore_axis_name="s")


@functools.cache
def _build_segsum():
    return functools.partial(
        pl.kernel,
        out_type=jax.ShapeDtypeStruct((NCB, NP, CB), f32),
        mesh=_mesh(),
        scratch_types=[
            pltpu.VMEM((2, SB, CHUNK), i32),
            pltpu.VMEM((2, SB, CHUNK), i32),
            pltpu.VMEM((2, SB, CHUNK), i32),
            pltpu.VMEM((2, CHUNK, CB), f32),
            pltpu.VMEM((WCHUNK, CB), f32),
            pltpu.VMEM((WCHUNK, CB), f32),
            pltpu.VMEM_SHARED((NP, CB), f32),
            pltpu.SemaphoreType.DMA,
            pltpu.SemaphoreType.DMA,
            pltpu.SemaphoreType.DMA,
        ],
        compiler_params=pltpu.CompilerParams(use_tc_tiling_on_sc=False),
    )(_segsum_body)


def _segsum(m2, gsrc2, sdst2):
    return _build_segsum()(m2, gsrc2, sdst2)


def _segsum_body(m2, gsrc2, sdst2, out, sidx, didx, ridx, rows, zbuf, obuf,
                 acc, gsem, isem, ssem):
    c = lax.axis_index("c")
    s = lax.axis_index("s")
    rowbase = s * TROWS
    rbase = s * ROWS_PER_TILE

    def _z(i, _):
        zbuf[i // 2, pl.ds((i % 2) * 16, 16)] = jnp.zeros((16,), f32)
        return _
    lax.fori_loop(0, WCHUNK * 2, _z, None)

    def idx_fire(i, u):
        r0 = jnp.minimum(rowbase + i * SB, EROWS - SB)
        pltpu.async_copy(gsrc2.at[pl.ds(r0, SB)], sidx.at[u], isem)
        pltpu.async_copy(sdst2.at[pl.ds(r0, SB)], didx.at[u], isem)

    def idx_wait(u):
        pltpu.make_async_copy(gsrc2.at[pl.ds(0, SB)], sidx.at[u], isem).wait()
        pltpu.make_async_copy(sdst2.at[pl.ds(0, SB)], didx.at[u], isem).wait()

    def ridx_compute(u, cbi):
        for k in range(SB):
            for q in range(CHUNK // 16):
                v = sidx[u, k, pl.ds(q * 16, 16)]
                ridx[u, k, pl.ds(q * 16, 16)] = v * NCB + cbi

    def gfire(u, k, p):
        pltpu.async_copy(m2.at[ridx.at[u, k]], rows.at[p], gsem)

    def gwait(p):
        pltpu.make_async_copy(m2.at[pl.ds(0, CHUNK)], rows.at[p], gsem).wait()

    def swait(p):
        pltpu.make_async_copy(m2.at[pl.ds(0, CHUNK)], rows.at[p], ssem).wait()

    def process_sblk(i, u, cbi, first=False):
        un = 1 - u

        @pl.when(i < NSB - 1)
        def _():
            idx_wait(un)
            ridx_compute(un, cbi)
        for k in range(SB):
            p = k % 2
            if not (first and k == 0):
                swait(1 - p)
            if k < SB - 1:
                gfire(u, k + 1, 1 - p)
            else:
                @pl.when(i < NSB - 1)
                def _():
                    gfire(un, 0, 1 - p)
            gwait(p)
            pltpu.async_copy(rows.at[p], acc.at[didx.at[u, k]], ssem,
                             add=True)

        @pl.when(i + 2 < NSB)
        def _():
            idx_fire(i + 2, u)

    def _pass(pp):
        cbi = c * 2 + pp

        def _zc(k, _):
            pltpu.sync_copy(zbuf, acc.at[pl.ds(rbase + k * WCHUNK, WCHUNK)])
            return _
        lax.fori_loop(0, ROWS_PER_TILE // WCHUNK, _zc, None)
        plsc.subcore_barrier()

        idx_fire(0, 0)
        idx_fire(1, 1)
        idx_wait(0)
        ridx_compute(0, cbi)
        gfire(0, 0, 0)
        process_sblk(jnp.int32(0), 0, cbi, first=True)

        def _sbpair(i2, _):
            process_sblk(2 * i2 + 1, 1, cbi)
            process_sblk(2 * i2 + 2, 0, cbi)
            return _
        lax.fori_loop(0, (NSB - 1) // 2, _sbpair, None)
        swait(1)
        plsc.subcore_barrier()

        def _wc(k, _):
            r0 = rbase + k * WCHUNK
            pltpu.sync_copy(acc.at[pl.ds(r0, WCHUNK)], obuf)
            pltpu.sync_copy(obuf, out.at[cbi, pl.ds(r0, WCHUNK)])
            return _
        lax.fori_loop(0, ROWS_PER_TILE // WCHUNK, _wc, None)
        plsc.subcore_barrier()

    _pass(0)
    _pass(1)


_SV_PER_W = 200
_SV_WORKERS = NSV // _SV_PER_W


@functools.cache
def _build_sv_gather():
    return functools.partial(
        pl.kernel,
        out_type=jax.ShapeDtypeStruct((NSV, DP), f32),
        mesh=_mesh(),
        scratch_types=[
            pltpu.VMEM((_SV_PER_W,), i32),
            pltpu.VMEM((_SV_PER_W, DP), f32),
            pltpu.SemaphoreType.DMA,
        ],
    )(_sv_gather_body)


def _sv_gather(vs, sv):
    return _build_sv_gather()(vs, sv)


def _sv_gather_body(vs, sv, out, idxb, rowsb, sem):
    c = lax.axis_index("c")
    s = lax.axis_index("s")
    w = s * NC + c

    @pl.when(w < _SV_WORKERS)
    def _():
        base = w * _SV_PER_W
        pltpu.sync_copy(sv.at[pl.ds(base, _SV_PER_W)], idxb)
        pltpu.async_copy(vs.at[idxb], rowsb, sem).wait()
        pltpu.sync_copy(rowsb, out.at[pl.ds(base, _SV_PER_W)])


def _pad_wt(w, ki=VHS, ko=VHS):
    return jnp.zeros((DP, DP), f32).at[:ki, :ko].set(w.T[:ki, :ko])


def _pad_b(b):
    return jnp.zeros((1, DP), f32).at[0, :b.shape[0]].set(b)


def _prep_mlp(p):
    return {"w1": _pad_wt(p["w1"]), "b1": _pad_b(p["b1"]),
            "w2": _pad_wt(p["w2"]), "b2": _pad_b(p["b2"]),
            "w3": _pad_wt(p["w3"]), "b3": _pad_b(p["b3"])}


def _prep_gru(p):
    wi = jnp.zeros((DP, 3 * DP), f32)
    wh = jnp.zeros((DP, 3 * DP), f32)
    bi = jnp.zeros((1, 3 * DP), f32)
    bh = jnp.zeros((1, 3 * DP), f32)
    for g in range(3):
        wi = wi.at[:VHS, g * DP:g * DP + VHS].set(
            p["wi"][g * VHS:(g + 1) * VHS].T)
        wh = wh.at[:VHS, g * DP:g * DP + VHS].set(
            p["wh"][g * VHS:(g + 1) * VHS].T)
    for g in range(2):
        bi = bi.at[0, g * DP:g * DP + VHS].set(
            p["bi"][g * VHS:(g + 1) * VHS] + p["bh"][g * VHS:(g + 1) * VHS])
    bi = bi.at[0, 2 * DP:2 * DP + VHS].set(p["bi"][2 * VHS:3 * VHS])
    bh = bh.at[0, 2 * DP:2 * DP + VHS].set(p["bh"][2 * VHS:3 * VHS])
    return {"wi": wi, "wh": wh, "bi": bi, "bh": bh}


def kernel(x, ind, outd, params, sv_node, edge_index):
    nvt = x.shape[1]
    tv = params["type_w"] + params["type_b"]
    tv_pad = jnp.zeros((DP, DP), f32).at[:nvt, :VHS].set(tv)
    x_pad = jnp.zeros((N, DP), f32).at[:, :nvt].set(x)

    fmsg = _prep_mlp(params["fmsg"])
    bmsg = _prep_mlp(params["bmsg"])
    fgru = _prep_gru(params["fgru"])
    bgru = _prep_gru(params["bgru"])
    pre = {"w1": _pad_wt(params["pre"]["w1"], ki=VHS, ko=30),
           "b1": _pad_b(params["pre"]["b1"]),
           "w2": _pad_wt(params["pre"]["w2"], ki=30, ko=1),
           "b2": _pad_b(params["pre"]["b2"])}

    src = edge_index[0]
    dst = edge_index[1]
    gpad = jnp.arange(EPAD, dtype=i32) % 1024
    spad = N + jnp.arange(EPAD, dtype=i32) % 128
    gsrc_f = jnp.concatenate([src, gpad]).reshape(EROWS, CHUNK)
    sdst_f = jnp.concatenate([dst, spad]).reshape(EROWS, CHUNK)
    gsrc_b = jnp.concatenate([dst, gpad]).reshape(EROWS, CHUNK)
    sdst_b = jnp.concatenate([src, spad]).reshape(EROWS, CHUNK)
    ind_c = ind[:, None]
    outd_c = outd[:, None]

    vs = _embed(x_pad, tv_pad)
    for _ in range(NROUNDS):
        m = _mlp(vs, fmsg)
        raw4 = _segsum(m.reshape(N * NCB, CB), gsrc_f, sdst_f)
        vs = _gru(raw4, ind_c, vs, fgru)
        m = _mlp(vs, bmsg)
        raw4 = _segsum(m.reshape(N * NCB, CB), gsrc_b, sdst_b)
        vs = _gru(raw4, outd_c, vs, bgru)

    shortcut = _sv_gather(vs, sv_node)
    out = _head(shortcut, pre)
    return out[:, 0]


# speedup vs baseline: 1.7358x; 1.3975x over previous
import functools

import jax
import jax.numpy as jnp
from jax import lax
from jax.experimental import pallas as pl
from jax.experimental.pallas import tpu as pltpu
from jax.experimental.pallas import tpu_sc as plsc

N = 50000
E = 800000
VHS = 100
DP = 128
CB = 64
NCB = DP // CB
NSV = 5000
NROUNDS = 4

BN = 1000
CHUNK = 128
NS = 16
NC = 2

f32 = jnp.float32
bf16 = jnp.bfloat16
i32 = jnp.int32

EROWS = 6272
EPAD = EROWS * CHUNK - E
TROWS = EROWS // NS
SB = 8
NSB = TROWS // SB
NP = 50176
ROWS_PER_TILE = NP // NS
WCHUNK = 112


def _full(shape):
    return pl.BlockSpec(shape, lambda i: (0,) * len(shape))


def _embed_body(x_ref, tv_ref, o_ref):
    o_ref[...] = jnp.dot(x_ref[...], tv_ref[...], preferred_element_type=f32)


def _embed(x_pad, tv):
    return pl.pallas_call(
        _embed_body,
        grid=(N // BN,),
        in_specs=[pl.BlockSpec((BN, DP), lambda i: (i, 0)), _full((DP, DP))],
        out_specs=pl.BlockSpec((BN, DP), lambda i: (i, 0)),
        out_shape=jax.ShapeDtypeStruct((N, DP), f32),
    )(x_pad, tv)


def _mlp_body(x_ref, w1, b1, w2, b2, w3, b3, o_ref):
    x = x_ref[...]
    h = jnp.maximum(jnp.dot(x, w1[...], preferred_element_type=f32) + b1[...], 0.0)
    h = jnp.maximum(jnp.dot(h, w2[...], preferred_element_type=f32) + b2[...], 0.0)
    o_ref[...] = (jnp.dot(h, w3[...], preferred_element_type=f32)
                  + b3[...]).astype(bf16)


def _mlp(xs, p):
    return pl.pallas_call(
        _mlp_body,
        grid=(N // BN,),
        in_specs=[pl.BlockSpec((BN, DP), lambda i: (i, 0)),
                  _full((DP, DP)), _full((1, DP)),
                  _full((DP, DP)), _full((1, DP)),
                  _full((DP, DP)), _full((1, DP))],
        out_specs=pl.BlockSpec((BN, DP), lambda i: (i, 0)),
        out_shape=jax.ShapeDtypeStruct((N, DP), bf16),
    )(xs, p["w1"], p["b1"], p["w2"], p["b2"], p["w3"], p["b3"])


def _gru_body(msg4_ref, den_ref, h_ref, wi_ref, wh_ref, bi_ref, bh_ref, o_ref):
    msg = jnp.concatenate([msg4_ref[0], msg4_ref[1]], axis=-1).astype(f32)
    xin = msg / den_ref[...]
    h = h_ref[...]
    gi = jnp.dot(xin, wi_ref[...], preferred_element_type=f32) + bi_ref[...]
    gh = jnp.dot(h, wh_ref[...], preferred_element_type=f32) + bh_ref[...]
    r = jax.nn.sigmoid(gi[:, 0:DP] + gh[:, 0:DP])
    z = jax.nn.sigmoid(gi[:, DP:2 * DP] + gh[:, DP:2 * DP])
    n = jnp.tanh(gi[:, 2 * DP:3 * DP] + r * gh[:, 2 * DP:3 * DP])
    o_ref[...] = (1.0 - z) * n + z * h


def _gru(msg4, den, h, p):
    return pl.pallas_call(
        _gru_body,
        grid=(N // BN,),
        in_specs=[pl.BlockSpec((NCB, BN, CB), lambda i: (0, i, 0)),
                  pl.BlockSpec((BN, 1), lambda i: (i, 0)),
                  pl.BlockSpec((BN, DP), lambda i: (i, 0)),
                  _full((DP, 3 * DP)), _full((DP, 3 * DP)),
                  _full((1, 3 * DP)), _full((1, 3 * DP))],
        out_specs=pl.BlockSpec((BN, DP), lambda i: (i, 0)),
        out_shape=jax.ShapeDtypeStruct((N, DP), f32),
    )(msg4, den, h, p["wi"], p["wh"], p["bi"], p["bh"])


def _head_body(x_ref, w1, b1, w2, b2, o_ref):
    h = jnp.maximum(jnp.dot(x_ref[...], w1[...], preferred_element_type=f32)
                    + b1[...], 0.0)
    o_ref[...] = jnp.dot(h, w2[...], preferred_element_type=f32) + b2[...]


def _head(xs, p):
    return pl.pallas_call(
        _head_body,
        grid=(NSV // BN,),
        in_specs=[pl.BlockSpec((BN, DP), lambda i: (i, 0)),
                  _full((DP, DP)), _full((1, DP)),
                  _full((DP, DP)), _full((1, DP))],
        out_specs=pl.BlockSpec((BN, DP), lambda i: (i, 0)),
        out_shape=jax.ShapeDtypeStruct((NSV, DP), f32),
    )(xs, p["w1"], p["b1"], p["w2"], p["b2"])


@functools.cache
def _mesh():
    return plsc.VectorSubcoreMesh(core_axis_name="c", subcore_axis_name="s")


@functools.cache
def _build_segsum():
    return functools.partial(
        pl.kernel,
        out_type=jax.ShapeDtypeStruct((NCB, NP, CB), bf16),
        mesh=_mesh(),
        scratch_types=[
            pltpu.VMEM((2, SB, CHUNK), i32),
            pltpu.VMEM((2, SB, CHUNK), i32),
            pltpu.VMEM((2, SB, CHUNK), i32),
            pltpu.VMEM((2, CHUNK, CB), bf16),
            pltpu.VMEM((WCHUNK, CB), bf16),
            pltpu.VMEM((WCHUNK, CB), bf16),
            pltpu.VMEM_SHARED((NP, CB), bf16),
            pltpu.SemaphoreType.DMA,
            pltpu.SemaphoreType.DMA,
            pltpu.SemaphoreType.DMA,
        ],
        compiler_params=pltpu.CompilerParams(use_tc_tiling_on_sc=False),
    )(_segsum_body)


def _segsum(m2, gsrc2, sdst2):
    return _build_segsum()(m2, gsrc2, sdst2)


def _segsum_body(m2, gsrc2, sdst2, out, sidx, didx, ridx, rows, zbuf, obuf,
                 acc, gsem, isem, ssem):
    c = lax.axis_index("c")
    s = lax.axis_index("s")
    rowbase = s * TROWS
    rbase = s * ROWS_PER_TILE

    def _z(i, _):
        zbuf[i // 2, pl.ds((i % 2) * 32, 32)] = jnp.zeros((32,), bf16)
        return _
    lax.fori_loop(0, WCHUNK * 2, _z, None)

    def idx_fire(i, u):
        r0 = jnp.minimum(rowbase + i * SB, EROWS - SB)
        pltpu.async_copy(gsrc2.at[pl.ds(r0, SB)], sidx.at[u], isem)
        pltpu.async_copy(sdst2.at[pl.ds(r0, SB)], didx.at[u], isem)

    def idx_wait(u):
        pltpu.make_async_copy(gsrc2.at[pl.ds(0, SB)], sidx.at[u], isem).wait()
        pltpu.make_async_copy(sdst2.at[pl.ds(0, SB)], didx.at[u], isem).wait()

    def ridx_compute(u, cbi):
        for k in range(SB):
            for q in range(CHUNK // 16):
                v = sidx[u, k, pl.ds(q * 16, 16)]
                ridx[u, k, pl.ds(q * 16, 16)] = v * NCB + cbi

    def gfire(u, k, p):
        pltpu.async_copy(m2.at[ridx.at[u, k]], rows.at[p], gsem)

    def gwait(p):
        pltpu.make_async_copy(m2.at[pl.ds(0, CHUNK)], rows.at[p], gsem).wait()

    def swait(p):
        pltpu.make_async_copy(m2.at[pl.ds(0, CHUNK)], rows.at[p], ssem).wait()

    def process_sblk(i, u, cbi, first=False):
        un = 1 - u

        @pl.when(i < NSB - 1)
        def _():
            idx_wait(un)
            ridx_compute(un, cbi)
        for k in range(SB):
            p = k % 2
            if not (first and k == 0):
                swait(1 - p)
            if k < SB - 1:
                gfire(u, k + 1, 1 - p)
            else:
                @pl.when(i < NSB - 1)
                def _():
                    gfire(un, 0, 1 - p)
            gwait(p)
            pltpu.async_copy(rows.at[p], acc.at[didx.at[u, k]], ssem,
                             add=True)

        @pl.when(i + 2 < NSB)
        def _():
            idx_fire(i + 2, u)

    def _pass(pp):
        cbi = c

        def _zc(k, _):
            pltpu.sync_copy(zbuf, acc.at[pl.ds(rbase + k * WCHUNK, WCHUNK)])
            return _
        lax.fori_loop(0, ROWS_PER_TILE // WCHUNK, _zc, None)
        plsc.subcore_barrier()

        idx_fire(0, 0)
        idx_fire(1, 1)
        idx_wait(0)
        ridx_compute(0, cbi)
        gfire(0, 0, 0)
        process_sblk(jnp.int32(0), 0, cbi, first=True)

        def _sbpair(i2, _):
            process_sblk(2 * i2 + 1, 1, cbi)
            process_sblk(2 * i2 + 2, 0, cbi)
            return _
        lax.fori_loop(0, (NSB - 1) // 2, _sbpair, None)
        swait(1)
        plsc.subcore_barrier()

        def _wc(k, _):
            r0 = rbase + k * WCHUNK
            pltpu.sync_copy(acc.at[pl.ds(r0, WCHUNK)], obuf)
            pltpu.sync_copy(obuf, out.at[cbi, pl.ds(r0, WCHUNK)])
            return _
        lax.fori_loop(0, ROWS_PER_TILE // WCHUNK, _wc, None)
        plsc.subcore_barrier()

    _pass(0)


_SV_PER_W = 200
_SV_WORKERS = NSV // _SV_PER_W


@functools.cache
def _build_sv_gather():
    return functools.partial(
        pl.kernel,
        out_type=jax.ShapeDtypeStruct((NSV, DP), f32),
        mesh=_mesh(),
        scratch_types=[
            pltpu.VMEM((_SV_PER_W,), i32),
            pltpu.VMEM((_SV_PER_W, DP), f32),
            pltpu.SemaphoreType.DMA,
        ],
    )(_sv_gather_body)


def _sv_gather(vs, sv):
    return _build_sv_gather()(vs, sv)


def _sv_gather_body(vs, sv, out, idxb, rowsb, sem):
    c = lax.axis_index("c")
    s = lax.axis_index("s")
    w = s * NC + c

    @pl.when(w < _SV_WORKERS)
    def _():
        base = w * _SV_PER_W
        pltpu.sync_copy(sv.at[pl.ds(base, _SV_PER_W)], idxb)
        pltpu.async_copy(vs.at[idxb], rowsb, sem).wait()
        pltpu.sync_copy(rowsb, out.at[pl.ds(base, _SV_PER_W)])


def _pad_wt(w, ki=VHS, ko=VHS):
    return jnp.zeros((DP, DP), f32).at[:ki, :ko].set(w.T[:ki, :ko])


def _pad_b(b):
    return jnp.zeros((1, DP), f32).at[0, :b.shape[0]].set(b)


def _prep_mlp(p):
    return {"w1": _pad_wt(p["w1"]), "b1": _pad_b(p["b1"]),
            "w2": _pad_wt(p["w2"]), "b2": _pad_b(p["b2"]),
            "w3": _pad_wt(p["w3"]), "b3": _pad_b(p["b3"])}


def _prep_gru(p):
    wi = jnp.zeros((DP, 3 * DP), f32)
    wh = jnp.zeros((DP, 3 * DP), f32)
    bi = jnp.zeros((1, 3 * DP), f32)
    bh = jnp.zeros((1, 3 * DP), f32)
    for g in range(3):
        wi = wi.at[:VHS, g * DP:g * DP + VHS].set(
            p["wi"][g * VHS:(g + 1) * VHS].T)
        wh = wh.at[:VHS, g * DP:g * DP + VHS].set(
            p["wh"][g * VHS:(g + 1) * VHS].T)
    for g in range(2):
        bi = bi.at[0, g * DP:g * DP + VHS].set(
            p["bi"][g * VHS:(g + 1) * VHS] + p["bh"][g * VHS:(g + 1) * VHS])
    bi = bi.at[0, 2 * DP:2 * DP + VHS].set(p["bi"][2 * VHS:3 * VHS])
    bh = bh.at[0, 2 * DP:2 * DP + VHS].set(p["bh"][2 * VHS:3 * VHS])
    return {"wi": wi, "wh": wh, "bi": bi, "bh": bh}


def kernel(x, ind, outd, params, sv_node, edge_index):
    nvt = x.shape[1]
    tv = params["type_w"] + params["type_b"]
    tv_pad = jnp.zeros((DP, DP), f32).at[:nvt, :VHS].set(tv)
    x_pad = jnp.zeros((N, DP), f32).at[:, :nvt].set(x)

    fmsg = _prep_mlp(params["fmsg"])
    bmsg = _prep_mlp(params["bmsg"])
    fgru = _prep_gru(params["fgru"])
    bgru = _prep_gru(params["bgru"])
    pre = {"w1": _pad_wt(params["pre"]["w1"], ki=VHS, ko=30),
           "b1": _pad_b(params["pre"]["b1"]),
           "w2": _pad_wt(params["pre"]["w2"], ki=30, ko=1),
           "b2": _pad_b(params["pre"]["b2"])}

    src = edge_index[0]
    dst = edge_index[1]
    gpad = jnp.arange(EPAD, dtype=i32) % 1024
    spad = N + jnp.arange(EPAD, dtype=i32) % 128
    gsrc_f = jnp.concatenate([src, gpad]).reshape(EROWS, CHUNK)
    sdst_f = jnp.concatenate([dst, spad]).reshape(EROWS, CHUNK)
    gsrc_b = jnp.concatenate([dst, gpad]).reshape(EROWS, CHUNK)
    sdst_b = jnp.concatenate([src, spad]).reshape(EROWS, CHUNK)
    ind_c = ind[:, None]
    outd_c = outd[:, None]

    vs = _embed(x_pad, tv_pad)
    for _ in range(NROUNDS):
        m = _mlp(vs, fmsg)
        raw4 = _segsum(m.reshape(N * NCB, CB), gsrc_f, sdst_f)
        vs = _gru(raw4, ind_c, vs, fgru)
        m = _mlp(vs, bmsg)
        raw4 = _segsum(m.reshape(N * NCB, CB), gsrc_b, sdst_b)
        vs = _gru(raw4, outd_c, vs, bgru)

    shortcut = _sv_gather(vs, sv_node)
    out = _head(shortcut, pre)
    return out[:, 0]
